# baseline (speedup 1.0000x reference)
import functools
import math

import jax
import jax.numpy as jnp
from jax import lax
from jax.experimental import pallas as pl
from jax.experimental.pallas import tpu as pltpu
from jax.experimental.pallas import tpu_sc as plsc

_N = 10000
_E = 160000
_D = 128

_NH = 4
_EH = _E // _NH
_BE = 1600
_NB = _EH // _BE
_NROW = 1000

_NW = 32
_C = 80
_NCHUNK = _EH // _C
_TRIPS = _NCHUNK // _NW
_XTRA = _NCHUNK - _TRIPS * _NW
_STRIPE = 624

_PREC = lax.Precision.DEFAULT


def _silu(v):
    return v * jax.nn.sigmoid(v)


def _rne_bits(x):
    u = lax.bitcast_convert_type(x, jnp.int32)
    return (u + 0x7FFF + ((u >> 16) & 1)) & jnp.int32(-65536)


def _pack2(lo, hi):
    return jnp.bitwise_or((_rne_bits(lo) >> 16) & 65535, _rne_bits(hi))


def _unpack_lo(u):
    return lax.bitcast_convert_type(u << 16, jnp.float32)


def _unpack_hi(u):
    return lax.bitcast_convert_type(u & jnp.int32(-65536), jnp.float32)


def _node_proj_body(x_ref, w_ref, p_ref, q_ref):
    y = jnp.dot(x_ref[...], w_ref[...], precision=_PREC,
                preferred_element_type=jnp.float32)
    p_ref[...] = _pack2(y[:, :256], y[:, 256:512])
    q_ref[...] = y[:, 512:]


def _pass_a_body(ef_ref, s_ref, q_ref, we_ref, b1_ref, kw2_ref, kb2_ref,
                 lw2_ref, lb2_ref, ao_ref, sum_ref, sq_ref):
    t = jnp.dot(ef_ref[...], we_ref[...], precision=_PREC,
                preferred_element_type=jnp.float32)
    su = s_ref[...]
    tk = t[:, :256] + _unpack_lo(su) + b1_ref[:, :256]
    tv = t[:, 256:] + _unpack_hi(su) + b1_ref[:, 256:]
    key = jnp.dot(_silu(tk), kw2_ref[...], precision=_PREC,
                  preferred_element_type=jnp.float32) + kb2_ref[...]
    q = q_ref[...]
    alpha = jnp.concatenate([q, q], axis=1) * key * (1.0 / 16.0)
    out = jnp.dot(_silu(tv), lw2_ref[...], precision=_PREC,
                  preferred_element_type=jnp.float32) + lb2_ref[...]
    ao_ref[...] = _pack2(alpha, out)

    @pl.when(pl.program_id(0) == 0)
    def _():
        sum_ref[...] = jnp.zeros_like(sum_ref)
        sq_ref[...] = jnp.zeros_like(sq_ref)

    sum_ref[...] += jnp.sum(alpha, axis=0, keepdims=True)
    sq_ref[...] += jnp.sum(alpha * alpha, axis=0, keepdims=True)


def _pass_b_body(inv_e, ao_ref, ss_ref, qq_ref, bng_ref,
                 bnb_ref, mlw_ref, mlb_ref, lng_ref, lnb_ref, out_ref):
    mean = jnp.sum(ss_ref[...], axis=0, keepdims=True) * inv_e
    var = jnp.sum(qq_ref[...], axis=0, keepdims=True) * inv_e - mean * mean
    s = bng_ref[...] * lax.rsqrt(var + 1e-5)
    t = bnb_ref[...] - mean * s
    ao = ao_ref[...]
    g = _unpack_hi(ao) * jax.nn.sigmoid(_unpack_lo(ao) * s + t)
    o = jnp.dot(g, mlw_ref[...], precision=_PREC,
                preferred_element_type=jnp.float32) + mlb_ref[...]
    mu = jnp.mean(o, axis=1, keepdims=True)
    v2 = jnp.mean(o * o, axis=1, keepdims=True) - mu * mu
    o = (o - mu) * lax.rsqrt(v2 + 1e-5) * lng_ref[...] + lnb_ref[...]
    out_ref[...] = _silu(o)


def _final_body(n, x_ref, a1_ref, a2_ref, a3_ref, a4_ref, lcw_ref, lcb_ref,
                g_ref, b_ref, out_ref):
    hv = ((a1_ref[:n, :] + a1_ref[n:, :]) + (a2_ref[:n, :] + a2_ref[n:, :])
          + (a3_ref[:n, :] + a3_ref[n:, :]) + (a4_ref[:n, :] + a4_ref[n:, :]))
    o = jnp.dot(hv, lcw_ref[...], precision=_PREC,
                preferred_element_type=jnp.float32) + lcb_ref[...]
    mean = jnp.mean(o, axis=0, keepdims=True)
    var = jnp.mean(o * o, axis=0, keepdims=True) - mean * mean
    o = (o - mean) * lax.rsqrt(var + 1e-5) * g_ref[...] + b_ref[...]
    out_ref[...] = x_ref[...] + _silu(o)


def _make_gather_body(e_off):
    def body_fn(src_hbm, dst_hbm, p_hbm, q_hbm, o1_hbm, o2_hbm,
                idx1, rows1, idx2, rows2):
        wid = lax.axis_index("c") * 16 + lax.axis_index("s")
        trips = _TRIPS + jnp.where(wid < _XTRA, 1, 0)

        def body(i, carry):
            obase = (wid + _NW * i) * _C
            ibase = e_off + obase
            pltpu.sync_copy(src_hbm.at[pl.ds(ibase, _C)], idx1)
            pltpu.sync_copy(dst_hbm.at[pl.ds(ibase, _C)], idx2)
            pltpu.sync_copy(p_hbm.at[idx1], rows1)
            pltpu.sync_copy(rows1, o1_hbm.at[pl.ds(obase, _C)])
            pltpu.sync_copy(q_hbm.at[idx2], rows2)
            pltpu.sync_copy(rows2, o2_hbm.at[pl.ds(obase, _C)])
            return carry

        lax.fori_loop(0, trips, body, 0)

    return body_fn


def _make_scatter_body(e_off):
    def body_fn(dst_hbm, rows_hbm, zero_hbm, out_hbm, acc, idxv, rowsv):
        c = lax.axis_index("c")
        s = lax.axis_index("s")
        wid = c * 16 + s
        start = s * _STRIPE

        @pl.when(s < 15)
        def _():
            pltpu.sync_copy(zero_hbm.at[pl.ds(start, _STRIPE)],
                            acc.at[pl.ds(start, _STRIPE)])

        @pl.when(s == 15)
        def _():
            pltpu.sync_copy(zero_hbm.at[pl.ds(15 * _STRIPE, _N - 15 * _STRIPE)],
                            acc.at[pl.ds(15 * _STRIPE, _N - 15 * _STRIPE)])

        plsc.subcore_barrier()
        trips = _TRIPS + jnp.where(wid < _XTRA, 1, 0)

        def body(i, carry):
            obase = (wid + _NW * i) * _C
            pltpu.sync_copy(dst_hbm.at[pl.ds(e_off + obase, _C)], idxv)
            pltpu.sync_copy(rows_hbm.at[pl.ds(obase, _C)], rowsv)
            pltpu.sync_copy(rowsv, acc.at[idxv], add=True)
            return carry

        lax.fori_loop(0, trips, body, 0)
        plsc.subcore_barrier()

        @pl.when(s < 15)
        def _():
            pltpu.sync_copy(acc.at[pl.ds(start, _STRIPE)],
                            out_hbm.at[pl.ds(c * _N + start, _STRIPE)])

        @pl.when(s == 15)
        def _():
            pltpu.sync_copy(
                acc.at[pl.ds(15 * _STRIPE, _N - 15 * _STRIPE)],
                out_hbm.at[pl.ds(c * _N + 15 * _STRIPE, _N - 15 * _STRIPE)])

    return body_fn


@functools.lru_cache(maxsize=None)
def _sc_kernels(e_off):
    mesh = plsc.VectorSubcoreMesh(core_axis_name="c", subcore_axis_name="s")
    gather = pl.kernel(
        _make_gather_body(e_off), mesh=mesh,
        out_type=[jax.ShapeDtypeStruct((_EH, 256), jnp.int32),
                  jax.ShapeDtypeStruct((_EH, _D), jnp.float32)],
        scratch_types=[pltpu.VMEM((_C,), jnp.int32),
                       pltpu.VMEM((_C, 256), jnp.int32),
                       pltpu.VMEM((_C,), jnp.int32),
                       pltpu.VMEM((_C, _D), jnp.float32)],
    )
    scatter = pl.kernel(
        _make_scatter_body(e_off), mesh=mesh,
        out_type=jax.ShapeDtypeStruct((2 * _N, _D), jnp.float32),
        scratch_types=[pltpu.VMEM_SHARED((_N, _D), jnp.float32),
                       pltpu.VMEM((_C,), jnp.int32),
                       pltpu.VMEM((_C, _D), jnp.float32)],
    )
    return gather, scatter


def _node_proj(x, w):
    return pl.pallas_call(
        _node_proj_body,
        grid=(_N // _NROW,),
        in_specs=[pl.BlockSpec((_NROW, _D), lambda i: (i, 0)),
                  pl.BlockSpec((_D, 640), lambda i: (0, 0))],
        out_specs=[pl.BlockSpec((_NROW, 256), lambda i: (i, 0)),
                   pl.BlockSpec((_NROW, _D), lambda i: (i, 0))],
        out_shape=[jax.ShapeDtypeStruct((_N, 256), jnp.int32),
                   jax.ShapeDtypeStruct((_N, _D), jnp.float32)],
    )(x, w)


def _pass_a(blk_off, ef, s, q, we, b1, kw2, kb2, lw2, lb2):
    return pl.pallas_call(
        _pass_a_body,
        grid=(_NB,),
        in_specs=[pl.BlockSpec((_BE, _D), lambda i: (blk_off + i, 0)),
                  pl.BlockSpec((_BE, 256), lambda i: (i, 0)),
                  pl.BlockSpec((_BE, _D), lambda i: (i, 0)),
                  pl.BlockSpec((_D, 512), lambda i: (0, 0)),
                  pl.BlockSpec((1, 512), lambda i: (0, 0)),
                  pl.BlockSpec((256, 256), lambda i: (0, 0)),
                  pl.BlockSpec((1, 256), lambda i: (0, 0)),
                  pl.BlockSpec((256, 256), lambda i: (0, 0)),
                  pl.BlockSpec((1, 256), lambda i: (0, 0))],
        out_specs=[pl.BlockSpec((_BE, 256), lambda i: (i, 0)),
                   pl.BlockSpec((1, 256), lambda i: (0, 0)),
                   pl.BlockSpec((1, 256), lambda i: (0, 0))],
        out_shape=[jax.ShapeDtypeStruct((_EH, 256), jnp.int32),
                   jax.ShapeDtypeStruct((1, 256), jnp.float32),
                   jax.ShapeDtypeStruct((1, 256), jnp.float32)],
    )(ef, s, q, we, b1, kw2, kb2, lw2, lb2)


def _pass_b(ao, ss, qq, bng, bnb, mlw, mlb, lng, lnb):
    return pl.pallas_call(
        functools.partial(_pass_b_body, 1.0 / _E),
        grid=(_NB,),
        in_specs=[pl.BlockSpec((_BE, 256), lambda i: (i, 0)),
                  pl.BlockSpec((_NH, 256), lambda i: (0, 0)),
                  pl.BlockSpec((_NH, 256), lambda i: (0, 0)),
                  pl.BlockSpec((1, 256), lambda i: (0, 0)),
                  pl.BlockSpec((1, 256), lambda i: (0, 0)),
                  pl.BlockSpec((256, _D), lambda i: (0, 0)),
                  pl.BlockSpec((1, _D), lambda i: (0, 0)),
                  pl.BlockSpec((1, _D), lambda i: (0, 0)),
                  pl.BlockSpec((1, _D), lambda i: (0, 0))],
        out_specs=pl.BlockSpec((_BE, _D), lambda i: (i, 0)),
        out_shape=jax.ShapeDtypeStruct((_EH, _D), jnp.float32),
    )(ao, ss, qq, bng, bnb, mlw, mlb, lng, lnb)


def _final(x, a1, a2, a3, a4, lcw, lcb, g, b):
    return pl.pallas_call(
        functools.partial(_final_body, _N),
        grid=(1,),
        in_specs=[pl.BlockSpec((_N, _D), lambda i: (0, 0)),
                  pl.BlockSpec((2 * _N, _D), lambda i: (0, 0)),
                  pl.BlockSpec((2 * _N, _D), lambda i: (0, 0)),
                  pl.BlockSpec((2 * _N, _D), lambda i: (0, 0)),
                  pl.BlockSpec((2 * _N, _D), lambda i: (0, 0)),
                  pl.BlockSpec((_D, _D), lambda i: (0, 0)),
                  pl.BlockSpec((1, _D), lambda i: (0, 0)),
                  pl.BlockSpec((1, _D), lambda i: (0, 0)),
                  pl.BlockSpec((1, _D), lambda i: (0, 0))],
        out_specs=pl.BlockSpec((_N, _D), lambda i: (0, 0)),
        out_shape=jax.ShapeDtypeStruct((_N, _D), jnp.float32),
    )(x, a1, a2, a3, a4, lcw, lcb, g, b)


def kernel(x, edge_index, edge_feature, params):
    p = params['heads'][0]
    src = edge_index[0]
    dst = edge_index[1]

    wk = p['K_v2v'] @ p['ku_W1'][:_D]
    wv = p['V_v2v'] @ p['lu_W1'][:_D]
    w_node = jnp.concatenate([wk, wv, p['K_v2v']], axis=1)
    we = jnp.concatenate([p['K_e2v'] @ p['ku_W1'][_D:],
                          p['V_e2v'] @ p['lu_W1'][_D:]], axis=1)
    b1 = jnp.concatenate([p['ku_b1'], p['lu_b1']])[None, :]
    zeros = jnp.zeros((_N, _D), jnp.float32)

    ptab, qtab = _node_proj(x, w_node)

    gathers = []
    for h in range(_NH):
        g, _ = _sc_kernels(h * _EH)
        gathers.append(g(src, dst, ptab, qtab))

    sums = []
    sqs = []
    aos = []
    for h in range(_NH):
        s_e, q_e = gathers[h]
        ao, ssum, ssq = _pass_a(
            h * (_EH // _BE), edge_feature, s_e, q_e, we, b1,
            p['ku_W2'], p['ku_b2'][None], p['lu_W2'], p['lu_b2'][None])
        aos.append(ao)
        sums.append(ssum)
        sqs.append(ssq)
    ss = jnp.concatenate(sums, axis=0)
    qq = jnp.concatenate(sqs, axis=0)

    aggs = []
    for h in range(_NH):
        out_e = _pass_b(aos[h], ss, qq,
                        p['bn_g'][None], p['bn_b'][None],
                        p['ml_W'], p['ml_b'][None],
                        p['ln_g'][None], p['ln_b'][None])
        _, sc_scatter = _sc_kernels(h * _EH)
        aggs.append(sc_scatter(dst, out_e, zeros))

    return _final(x, aggs[0], aggs[1], aggs[2], aggs[3],
                  params['lc_W'], params['lc_b'][None],
                  params['bnv_g'][None], params['bnv_b'][None])


# speedup vs baseline: 1.0848x; 1.0848x over previous
import functools
import math

import jax
import jax.numpy as jnp
from jax import lax
from jax.experimental import pallas as pl
from jax.experimental.pallas import tpu as pltpu
from jax.experimental.pallas import tpu_sc as plsc

_N = 10000
_E = 160000
_D = 128

_NH = 2
_EH = _E // _NH
_BE = 1600
_NB = _EH // _BE
_NROW = 1000

_NW = 32
_C = 128
_NCHUNK = _EH // _C
_TRIPS = _NCHUNK // _NW
_XTRA = _NCHUNK - _TRIPS * _NW
_STRIPE = 624

_PREC = lax.Precision.DEFAULT


def _silu(v):
    return v * jax.nn.sigmoid(v)


def _rne_bits(x):
    u = lax.bitcast_convert_type(x, jnp.int32)
    return (u + 0x7FFF + ((u >> 16) & 1)) & jnp.int32(-65536)


def _pack2(lo, hi):
    return jnp.bitwise_or((_rne_bits(lo) >> 16) & 65535, _rne_bits(hi))


def _unpack_lo(u):
    return lax.bitcast_convert_type(u << 16, jnp.float32)


def _unpack_hi(u):
    return lax.bitcast_convert_type(u & jnp.int32(-65536), jnp.float32)


def _node_proj_body(x_ref, w_ref, p_ref, q_ref):
    y = jnp.dot(x_ref[...], w_ref[...], precision=_PREC,
                preferred_element_type=jnp.float32)
    p_ref[...] = _pack2(y[:, :256], y[:, 256:512])
    q_ref[...] = y[:, 512:]


def _pass_a_body(ef_ref, s_ref, q_ref, we_ref, b1_ref, kw2_ref, kb2_ref,
                 lw2_ref, lb2_ref, ao_ref, sum_ref, sq_ref):
    t = jnp.dot(ef_ref[...], we_ref[...], precision=_PREC,
                preferred_element_type=jnp.float32)
    su = s_ref[...]
    tk = t[:, :256] + _unpack_lo(su) + b1_ref[:, :256]
    tv = t[:, 256:] + _unpack_hi(su) + b1_ref[:, 256:]
    key = jnp.dot(_silu(tk), kw2_ref[...], precision=_PREC,
                  preferred_element_type=jnp.float32) + kb2_ref[...]
    q = q_ref[...]
    alpha = jnp.concatenate([q, q], axis=1) * key * (1.0 / 16.0)
    out = jnp.dot(_silu(tv), lw2_ref[...], precision=_PREC,
                  preferred_element_type=jnp.float32) + lb2_ref[...]
    ao_ref[...] = _pack2(alpha, out)

    @pl.when(pl.program_id(0) == 0)
    def _():
        sum_ref[...] = jnp.zeros_like(sum_ref)
        sq_ref[...] = jnp.zeros_like(sq_ref)

    sum_ref[...] += jnp.sum(alpha, axis=0, keepdims=True)
    sq_ref[...] += jnp.sum(alpha * alpha, axis=0, keepdims=True)


def _pass_b_body(inv_e, ao_ref, ss_ref, qq_ref, bng_ref,
                 bnb_ref, mlw_ref, mlb_ref, lng_ref, lnb_ref, out_ref):
    mean = jnp.sum(ss_ref[...], axis=0, keepdims=True) * inv_e
    var = jnp.sum(qq_ref[...], axis=0, keepdims=True) * inv_e - mean * mean
    s = bng_ref[...] * lax.rsqrt(var + 1e-5)
    t = bnb_ref[...] - mean * s
    ao = ao_ref[...]
    g = _unpack_hi(ao) * jax.nn.sigmoid(_unpack_lo(ao) * s + t)
    o = jnp.dot(g, mlw_ref[...], precision=_PREC,
                preferred_element_type=jnp.float32) + mlb_ref[...]
    mu = jnp.mean(o, axis=1, keepdims=True)
    v2 = jnp.mean(o * o, axis=1, keepdims=True) - mu * mu
    o = (o - mu) * lax.rsqrt(v2 + 1e-5) * lng_ref[...] + lnb_ref[...]
    out_ref[...] = _silu(o)


def _final_body(n, x_ref, a1_ref, a2_ref, lcw_ref, lcb_ref,
                g_ref, b_ref, out_ref):
    hv = (a1_ref[:n, :] + a1_ref[n:, :]) + (a2_ref[:n, :] + a2_ref[n:, :])
    o = jnp.dot(hv, lcw_ref[...], precision=_PREC,
                preferred_element_type=jnp.float32) + lcb_ref[...]
    mean = jnp.mean(o, axis=0, keepdims=True)
    var = jnp.mean(o * o, axis=0, keepdims=True) - mean * mean
    o = (o - mean) * lax.rsqrt(var + 1e-5) * g_ref[...] + b_ref[...]
    out_ref[...] = x_ref[...] + _silu(o)


def _make_gather_body(e_off):
    def body_fn(src_hbm, dst_hbm, p_hbm, q_hbm, o1_hbm, o2_hbm,
                idx1, rows1, idx2, rows2):
        wid = lax.axis_index("c") * 16 + lax.axis_index("s")
        trips = _TRIPS + jnp.where(wid < _XTRA, 1, 0)

        def body(i, carry):
            obase = (wid + _NW * i) * _C
            ibase = e_off + obase
            pltpu.sync_copy(src_hbm.at[pl.ds(ibase, _C)], idx1)
            pltpu.sync_copy(dst_hbm.at[pl.ds(ibase, _C)], idx2)
            pltpu.sync_copy(p_hbm.at[idx1], rows1)
            pltpu.sync_copy(rows1, o1_hbm.at[pl.ds(obase, _C)])
            pltpu.sync_copy(q_hbm.at[idx2], rows2)
            pltpu.sync_copy(rows2, o2_hbm.at[pl.ds(obase, _C)])
            return carry

        lax.fori_loop(0, trips, body, 0)

    return body_fn


def _make_scatter_body(e_off):
    def body_fn(dst_hbm, rows_hbm, zero_hbm, out_hbm, acc, idxv, rowsv):
        c = lax.axis_index("c")
        s = lax.axis_index("s")
        wid = c * 16 + s
        start = s * _STRIPE

        @pl.when(s < 15)
        def _():
            pltpu.sync_copy(zero_hbm.at[pl.ds(start, _STRIPE)],
                            acc.at[pl.ds(start, _STRIPE)])

        @pl.when(s == 15)
        def _():
            pltpu.sync_copy(zero_hbm.at[pl.ds(15 * _STRIPE, _N - 15 * _STRIPE)],
                            acc.at[pl.ds(15 * _STRIPE, _N - 15 * _STRIPE)])

        plsc.subcore_barrier()
        trips = _TRIPS + jnp.where(wid < _XTRA, 1, 0)

        def body(i, carry):
            obase = (wid + _NW * i) * _C
            pltpu.sync_copy(dst_hbm.at[pl.ds(e_off + obase, _C)], idxv)
            pltpu.sync_copy(rows_hbm.at[pl.ds(obase, _C)], rowsv)
            pltpu.sync_copy(rowsv, acc.at[idxv], add=True)
            return carry

        lax.fori_loop(0, trips, body, 0)
        plsc.subcore_barrier()

        @pl.when(s < 15)
        def _():
            pltpu.sync_copy(acc.at[pl.ds(start, _STRIPE)],
                            out_hbm.at[pl.ds(c * _N + start, _STRIPE)])

        @pl.when(s == 15)
        def _():
            pltpu.sync_copy(
                acc.at[pl.ds(15 * _STRIPE, _N - 15 * _STRIPE)],
                out_hbm.at[pl.ds(c * _N + 15 * _STRIPE, _N - 15 * _STRIPE)])

    return body_fn


@functools.lru_cache(maxsize=None)
def _sc_kernels(e_off):
    mesh = plsc.VectorSubcoreMesh(core_axis_name="c", subcore_axis_name="s")
    gather = pl.kernel(
        _make_gather_body(e_off), mesh=mesh,
        out_type=[jax.ShapeDtypeStruct((_EH, 256), jnp.int32),
                  jax.ShapeDtypeStruct((_EH, _D), jnp.float32)],
        scratch_types=[pltpu.VMEM((_C,), jnp.int32),
                       pltpu.VMEM((_C, 256), jnp.int32),
                       pltpu.VMEM((_C,), jnp.int32),
                       pltpu.VMEM((_C, _D), jnp.float32)],
    )
    scatter = pl.kernel(
        _make_scatter_body(e_off), mesh=mesh,
        out_type=jax.ShapeDtypeStruct((2 * _N, _D), jnp.float32),
        scratch_types=[pltpu.VMEM_SHARED((_N, _D), jnp.float32),
                       pltpu.VMEM((_C,), jnp.int32),
                       pltpu.VMEM((_C, _D), jnp.float32)],
    )
    return gather, scatter


def _node_proj(x, w):
    return pl.pallas_call(
        _node_proj_body,
        grid=(_N // _NROW,),
        in_specs=[pl.BlockSpec((_NROW, _D), lambda i: (i, 0)),
                  pl.BlockSpec((_D, 640), lambda i: (0, 0))],
        out_specs=[pl.BlockSpec((_NROW, 256), lambda i: (i, 0)),
                   pl.BlockSpec((_NROW, _D), lambda i: (i, 0))],
        out_shape=[jax.ShapeDtypeStruct((_N, 256), jnp.int32),
                   jax.ShapeDtypeStruct((_N, _D), jnp.float32)],
    )(x, w)


def _pass_a(blk_off, ef, s, q, we, b1, kw2, kb2, lw2, lb2):
    return pl.pallas_call(
        _pass_a_body,
        grid=(_NB,),
        in_specs=[pl.BlockSpec((_BE, _D), lambda i: (blk_off + i, 0)),
                  pl.BlockSpec((_BE, 256), lambda i: (i, 0)),
                  pl.BlockSpec((_BE, _D), lambda i: (i, 0)),
                  pl.BlockSpec((_D, 512), lambda i: (0, 0)),
                  pl.BlockSpec((1, 512), lambda i: (0, 0)),
                  pl.BlockSpec((256, 256), lambda i: (0, 0)),
                  pl.BlockSpec((1, 256), lambda i: (0, 0)),
                  pl.BlockSpec((256, 256), lambda i: (0, 0)),
                  pl.BlockSpec((1, 256), lambda i: (0, 0))],
        out_specs=[pl.BlockSpec((_BE, 256), lambda i: (i, 0)),
                   pl.BlockSpec((1, 256), lambda i: (0, 0)),
                   pl.BlockSpec((1, 256), lambda i: (0, 0))],
        out_shape=[jax.ShapeDtypeStruct((_EH, 256), jnp.int32),
                   jax.ShapeDtypeStruct((1, 256), jnp.float32),
                   jax.ShapeDtypeStruct((1, 256), jnp.float32)],
    )(ef, s, q, we, b1, kw2, kb2, lw2, lb2)


def _pass_b(ao, ss, qq, bng, bnb, mlw, mlb, lng, lnb):
    return pl.pallas_call(
        functools.partial(_pass_b_body, 1.0 / _E),
        grid=(_NB,),
        in_specs=[pl.BlockSpec((_BE, 256), lambda i: (i, 0)),
                  pl.BlockSpec((_NH, 256), lambda i: (0, 0)),
                  pl.BlockSpec((_NH, 256), lambda i: (0, 0)),
                  pl.BlockSpec((1, 256), lambda i: (0, 0)),
                  pl.BlockSpec((1, 256), lambda i: (0, 0)),
                  pl.BlockSpec((256, _D), lambda i: (0, 0)),
                  pl.BlockSpec((1, _D), lambda i: (0, 0)),
                  pl.BlockSpec((1, _D), lambda i: (0, 0)),
                  pl.BlockSpec((1, _D), lambda i: (0, 0))],
        out_specs=pl.BlockSpec((_BE, _D), lambda i: (i, 0)),
        out_shape=jax.ShapeDtypeStruct((_EH, _D), jnp.float32),
    )(ao, ss, qq, bng, bnb, mlw, mlb, lng, lnb)


def _final(x, a1, a2, lcw, lcb, g, b):
    return pl.pallas_call(
        functools.partial(_final_body, _N),
        grid=(1,),
        in_specs=[pl.BlockSpec((_N, _D), lambda i: (0, 0)),
                  pl.BlockSpec((2 * _N, _D), lambda i: (0, 0)),
                  pl.BlockSpec((2 * _N, _D), lambda i: (0, 0)),
                  pl.BlockSpec((_D, _D), lambda i: (0, 0)),
                  pl.BlockSpec((1, _D), lambda i: (0, 0)),
                  pl.BlockSpec((1, _D), lambda i: (0, 0)),
                  pl.BlockSpec((1, _D), lambda i: (0, 0))],
        out_specs=pl.BlockSpec((_N, _D), lambda i: (0, 0)),
        out_shape=jax.ShapeDtypeStruct((_N, _D), jnp.float32),
    )(x, a1, a2, lcw, lcb, g, b)


def kernel(x, edge_index, edge_feature, params):
    p = params['heads'][0]
    src = edge_index[0]
    dst = edge_index[1]

    wk = p['K_v2v'] @ p['ku_W1'][:_D]
    wv = p['V_v2v'] @ p['lu_W1'][:_D]
    w_node = jnp.concatenate([wk, wv, p['K_v2v']], axis=1)
    we = jnp.concatenate([p['K_e2v'] @ p['ku_W1'][_D:],
                          p['V_e2v'] @ p['lu_W1'][_D:]], axis=1)
    b1 = jnp.concatenate([p['ku_b1'], p['lu_b1']])[None, :]
    zeros = jnp.zeros((_N, _D), jnp.float32)

    ptab, qtab = _node_proj(x, w_node)

    gathers = []
    for h in range(_NH):
        g, _ = _sc_kernels(h * _EH)
        gathers.append(g(src, dst, ptab, qtab))

    sums = []
    sqs = []
    aos = []
    for h in range(_NH):
        s_e, q_e = gathers[h]
        ao, ssum, ssq = _pass_a(
            h * (_EH // _BE), edge_feature, s_e, q_e, we, b1,
            p['ku_W2'], p['ku_b2'][None], p['lu_W2'], p['lu_b2'][None])
        aos.append(ao)
        sums.append(ssum)
        sqs.append(ssq)
    ss = jnp.concatenate(sums, axis=0)
    qq = jnp.concatenate(sqs, axis=0)

    aggs = []
    for h in range(_NH):
        out_e = _pass_b(aos[h], ss, qq,
                        p['bn_g'][None], p['bn_b'][None],
                        p['ml_W'], p['ml_b'][None],
                        p['ln_g'][None], p['ln_b'][None])
        _, sc_scatter = _sc_kernels(h * _EH)
        aggs.append(sc_scatter(dst, out_e, zeros))

    return _final(x, aggs[0], aggs[1],
                  params['lc_W'], params['lc_b'][None],
                  params['bnv_g'][None], params['bnv_b'][None])


# speedup vs baseline: 1.1612x; 1.0704x over previous
import functools
import math

import jax
import jax.numpy as jnp
from jax import lax
from jax.experimental import pallas as pl
from jax.experimental.pallas import tpu as pltpu
from jax.experimental.pallas import tpu_sc as plsc

_N = 10000
_E = 160000
_D = 128

_NH = 2
_EH = _E // _NH
_BE = 1600
_NB = _EH // _BE
_NROW = 1000

_NW = 32
_C = 128
_NCHUNK = _EH // _C
_TRIPS = _NCHUNK // _NW
_XTRA = _NCHUNK - _TRIPS * _NW
_STRIPE = 624

_PREC = lax.Precision.DEFAULT


def _silu(v):
    return v * jax.nn.sigmoid(v)


def _rne_bits(x):
    u = lax.bitcast_convert_type(x, jnp.int32)
    return (u + 0x7FFF + ((u >> 16) & 1)) & jnp.int32(-65536)


def _pack2(lo, hi):
    return jnp.bitwise_or((_rne_bits(lo) >> 16) & 65535, _rne_bits(hi))


def _unpack_lo(u):
    return lax.bitcast_convert_type(u << 16, jnp.float32)


def _unpack_hi(u):
    return lax.bitcast_convert_type(u & jnp.int32(-65536), jnp.float32)


def _node_proj_body(x_ref, w_ref, p_ref, q_ref):
    y = jnp.dot(x_ref[...], w_ref[...], precision=_PREC,
                preferred_element_type=jnp.float32)
    p_ref[...] = _pack2(y[:, :256], y[:, 256:512])
    q_ref[...] = y[:, 512:]


def _pass_a_body(ef_ref, s_ref, q_ref, we_ref, b1_ref, kw2_ref, kb2_ref,
                 lw2_ref, lb2_ref, ao_ref, sum_ref, sq_ref):
    t = jnp.dot(ef_ref[...], we_ref[...], precision=_PREC,
                preferred_element_type=jnp.float32)
    su = s_ref[...]
    tk = t[:, :256] + _unpack_lo(su) + b1_ref[:, :256]
    tv = t[:, 256:] + _unpack_hi(su) + b1_ref[:, 256:]
    key = jnp.dot(_silu(tk), kw2_ref[...], precision=_PREC,
                  preferred_element_type=jnp.float32) + kb2_ref[...]
    q = q_ref[...]
    alpha = jnp.concatenate([q, q], axis=1) * key * (1.0 / 16.0)
    out = jnp.dot(_silu(tv), lw2_ref[...], precision=_PREC,
                  preferred_element_type=jnp.float32) + lb2_ref[...]
    ao_ref[...] = _pack2(alpha, out)

    @pl.when(pl.program_id(0) == 0)
    def _():
        sum_ref[...] = jnp.zeros_like(sum_ref)
        sq_ref[...] = jnp.zeros_like(sq_ref)

    sum_ref[...] += jnp.sum(alpha, axis=0, keepdims=True)
    sq_ref[...] += jnp.sum(alpha * alpha, axis=0, keepdims=True)


def _pass_b_body(inv_e, ao_ref, ss_ref, qq_ref, bng_ref,
                 bnb_ref, mlw_ref, mlb_ref, lng_ref, lnb_ref, out_ref):
    mean = jnp.sum(ss_ref[...], axis=0, keepdims=True) * inv_e
    var = jnp.sum(qq_ref[...], axis=0, keepdims=True) * inv_e - mean * mean
    s = bng_ref[...] * lax.rsqrt(var + 1e-5)
    t = bnb_ref[...] - mean * s
    ao = ao_ref[...]
    g = _unpack_hi(ao) * jax.nn.sigmoid(_unpack_lo(ao) * s + t)
    o = jnp.dot(g, mlw_ref[...], precision=_PREC,
                preferred_element_type=jnp.float32) + mlb_ref[...]
    mu = jnp.mean(o, axis=1, keepdims=True)
    v2 = jnp.mean(o * o, axis=1, keepdims=True) - mu * mu
    o = (o - mu) * lax.rsqrt(v2 + 1e-5) * lng_ref[...] + lnb_ref[...]
    out_ref[...] = _silu(o)


def _final_body(n, x_ref, a1_ref, a2_ref, lcw_ref, lcb_ref,
                g_ref, b_ref, out_ref):
    hv = (a1_ref[:n, :] + a1_ref[n:, :]) + (a2_ref[:n, :] + a2_ref[n:, :])
    o = jnp.dot(hv, lcw_ref[...], precision=_PREC,
                preferred_element_type=jnp.float32) + lcb_ref[...]
    mean = jnp.mean(o, axis=0, keepdims=True)
    var = jnp.mean(o * o, axis=0, keepdims=True) - mean * mean
    o = (o - mean) * lax.rsqrt(var + 1e-5) * g_ref[...] + b_ref[...]
    out_ref[...] = x_ref[...] + _silu(o)


def _make_gather_body(e_off):
    def body_fn(src_hbm, dst_hbm, p_hbm, q_hbm, o1_hbm, o2_hbm,
                idx1, rows1, idx2, rows2, sem1, sem2):
        wid = lax.axis_index("c") * 16 + lax.axis_index("s")
        trips = _TRIPS + jnp.where(wid < _XTRA, 1, 0)

        def body(i, carry):
            obase = (wid + _NW * i) * _C
            ibase = e_off + obase
            h1 = pltpu.async_copy(src_hbm.at[pl.ds(ibase, _C)], idx1, sem1)
            h2 = pltpu.async_copy(dst_hbm.at[pl.ds(ibase, _C)], idx2, sem2)
            h1.wait()
            h2.wait()
            g1 = pltpu.async_copy(p_hbm.at[idx1], rows1, sem1)
            g2 = pltpu.async_copy(q_hbm.at[idx2], rows2, sem2)
            g1.wait()
            g2.wait()
            w1 = pltpu.async_copy(rows1, o1_hbm.at[pl.ds(obase, _C)], sem1)
            w2 = pltpu.async_copy(rows2, o2_hbm.at[pl.ds(obase, _C)], sem2)
            w1.wait()
            w2.wait()
            return carry

        lax.fori_loop(0, trips, body, 0)

    return body_fn


def _make_scatter_body(e_off):
    def body_fn(dst_hbm, rows_hbm, zero_hbm, out_hbm, acc, idxv, rowsv,
                sem1, sem2):
        c = lax.axis_index("c")
        s = lax.axis_index("s")
        wid = c * 16 + s
        start = s * _STRIPE

        @pl.when(s < 15)
        def _():
            pltpu.sync_copy(zero_hbm.at[pl.ds(start, _STRIPE)],
                            acc.at[pl.ds(start, _STRIPE)])

        @pl.when(s == 15)
        def _():
            pltpu.sync_copy(zero_hbm.at[pl.ds(15 * _STRIPE, _N - 15 * _STRIPE)],
                            acc.at[pl.ds(15 * _STRIPE, _N - 15 * _STRIPE)])

        plsc.subcore_barrier()
        trips = _TRIPS + jnp.where(wid < _XTRA, 1, 0)

        def body(i, carry):
            obase = (wid + _NW * i) * _C
            h1 = pltpu.async_copy(dst_hbm.at[pl.ds(e_off + obase, _C)], idxv,
                                  sem1)
            h2 = pltpu.async_copy(rows_hbm.at[pl.ds(obase, _C)], rowsv, sem2)
            h1.wait()
            h2.wait()
            pltpu.sync_copy(rowsv, acc.at[idxv], add=True)
            return carry

        lax.fori_loop(0, trips, body, 0)
        plsc.subcore_barrier()

        @pl.when(s < 15)
        def _():
            pltpu.sync_copy(acc.at[pl.ds(start, _STRIPE)],
                            out_hbm.at[pl.ds(c * _N + start, _STRIPE)])

        @pl.when(s == 15)
        def _():
            pltpu.sync_copy(
                acc.at[pl.ds(15 * _STRIPE, _N - 15 * _STRIPE)],
                out_hbm.at[pl.ds(c * _N + 15 * _STRIPE, _N - 15 * _STRIPE)])

    return body_fn


@functools.lru_cache(maxsize=None)
def _sc_kernels(e_off):
    mesh = plsc.VectorSubcoreMesh(core_axis_name="c", subcore_axis_name="s")
    gather = pl.kernel(
        _make_gather_body(e_off), mesh=mesh,
        out_type=[jax.ShapeDtypeStruct((_EH, 256), jnp.int32),
                  jax.ShapeDtypeStruct((_EH, _D), jnp.float32)],
        scratch_types=[pltpu.VMEM((_C,), jnp.int32),
                       pltpu.VMEM((_C, 256), jnp.int32),
                       pltpu.VMEM((_C,), jnp.int32),
                       pltpu.VMEM((_C, _D), jnp.float32),
                       pltpu.SemaphoreType.DMA,
                       pltpu.SemaphoreType.DMA],
    )
    scatter = pl.kernel(
        _make_scatter_body(e_off), mesh=mesh,
        out_type=jax.ShapeDtypeStruct((2 * _N, _D), jnp.float32),
        scratch_types=[pltpu.VMEM_SHARED((_N, _D), jnp.float32),
                       pltpu.VMEM((_C,), jnp.int32),
                       pltpu.VMEM((_C, _D), jnp.float32),
                       pltpu.SemaphoreType.DMA,
                       pltpu.SemaphoreType.DMA],
    )
    return gather, scatter


def _node_proj(x, w):
    return pl.pallas_call(
        _node_proj_body,
        grid=(_N // _NROW,),
        in_specs=[pl.BlockSpec((_NROW, _D), lambda i: (i, 0)),
                  pl.BlockSpec((_D, 640), lambda i: (0, 0))],
        out_specs=[pl.BlockSpec((_NROW, 256), lambda i: (i, 0)),
                   pl.BlockSpec((_NROW, _D), lambda i: (i, 0))],
        out_shape=[jax.ShapeDtypeStruct((_N, 256), jnp.int32),
                   jax.ShapeDtypeStruct((_N, _D), jnp.float32)],
    )(x, w)


def _pass_a(blk_off, ef, s, q, we, b1, kw2, kb2, lw2, lb2):
    return pl.pallas_call(
        _pass_a_body,
        grid=(_NB,),
        in_specs=[pl.BlockSpec((_BE, _D), lambda i: (blk_off + i, 0)),
                  pl.BlockSpec((_BE, 256), lambda i: (i, 0)),
                  pl.BlockSpec((_BE, _D), lambda i: (i, 0)),
                  pl.BlockSpec((_D, 512), lambda i: (0, 0)),
                  pl.BlockSpec((1, 512), lambda i: (0, 0)),
                  pl.BlockSpec((256, 256), lambda i: (0, 0)),
                  pl.BlockSpec((1, 256), lambda i: (0, 0)),
                  pl.BlockSpec((256, 256), lambda i: (0, 0)),
                  pl.BlockSpec((1, 256), lambda i: (0, 0))],
        out_specs=[pl.BlockSpec((_BE, 256), lambda i: (i, 0)),
                   pl.BlockSpec((1, 256), lambda i: (0, 0)),
                   pl.BlockSpec((1, 256), lambda i: (0, 0))],
        out_shape=[jax.ShapeDtypeStruct((_EH, 256), jnp.int32),
                   jax.ShapeDtypeStruct((1, 256), jnp.float32),
                   jax.ShapeDtypeStruct((1, 256), jnp.float32)],
    )(ef, s, q, we, b1, kw2, kb2, lw2, lb2)


def _pass_b(ao, ss, qq, bng, bnb, mlw, mlb, lng, lnb):
    return pl.pallas_call(
        functools.partial(_pass_b_body, 1.0 / _E),
        grid=(_NB,),
        in_specs=[pl.BlockSpec((_BE, 256), lambda i: (i, 0)),
                  pl.BlockSpec((_NH, 256), lambda i: (0, 0)),
                  pl.BlockSpec((_NH, 256), lambda i: (0, 0)),
                  pl.BlockSpec((1, 256), lambda i: (0, 0)),
                  pl.BlockSpec((1, 256), lambda i: (0, 0)),
                  pl.BlockSpec((256, _D), lambda i: (0, 0)),
                  pl.BlockSpec((1, _D), lambda i: (0, 0)),
                  pl.BlockSpec((1, _D), lambda i: (0, 0)),
                  pl.BlockSpec((1, _D), lambda i: (0, 0))],
        out_specs=pl.BlockSpec((_BE, _D), lambda i: (i, 0)),
        out_shape=jax.ShapeDtypeStruct((_EH, _D), jnp.float32),
    )(ao, ss, qq, bng, bnb, mlw, mlb, lng, lnb)


def _final(x, a1, a2, lcw, lcb, g, b):
    return pl.pallas_call(
        functools.partial(_final_body, _N),
        grid=(1,),
        in_specs=[pl.BlockSpec((_N, _D), lambda i: (0, 0)),
                  pl.BlockSpec((2 * _N, _D), lambda i: (0, 0)),
                  pl.BlockSpec((2 * _N, _D), lambda i: (0, 0)),
                  pl.BlockSpec((_D, _D), lambda i: (0, 0)),
                  pl.BlockSpec((1, _D), lambda i: (0, 0)),
                  pl.BlockSpec((1, _D), lambda i: (0, 0)),
                  pl.BlockSpec((1, _D), lambda i: (0, 0))],
        out_specs=pl.BlockSpec((_N, _D), lambda i: (0, 0)),
        out_shape=jax.ShapeDtypeStruct((_N, _D), jnp.float32),
    )(x, a1, a2, lcw, lcb, g, b)


def kernel(x, edge_index, edge_feature, params):
    p = params['heads'][0]
    src = edge_index[0]
    dst = edge_index[1]

    wk = p['K_v2v'] @ p['ku_W1'][:_D]
    wv = p['V_v2v'] @ p['lu_W1'][:_D]
    w_node = jnp.concatenate([wk, wv, p['K_v2v']], axis=1)
    we = jnp.concatenate([p['K_e2v'] @ p['ku_W1'][_D:],
                          p['V_e2v'] @ p['lu_W1'][_D:]], axis=1)
    b1 = jnp.concatenate([p['ku_b1'], p['lu_b1']])[None, :]
    zeros = jnp.zeros((_N, _D), jnp.float32)

    ptab, qtab = _node_proj(x, w_node)

    gathers = []
    for h in range(_NH):
        g, _ = _sc_kernels(h * _EH)
        gathers.append(g(src, dst, ptab, qtab))

    sums = []
    sqs = []
    aos = []
    for h in range(_NH):
        s_e, q_e = gathers[h]
        ao, ssum, ssq = _pass_a(
            h * (_EH // _BE), edge_feature, s_e, q_e, we, b1,
            p['ku_W2'], p['ku_b2'][None], p['lu_W2'], p['lu_b2'][None])
        aos.append(ao)
        sums.append(ssum)
        sqs.append(ssq)
    ss = jnp.concatenate(sums, axis=0)
    qq = jnp.concatenate(sqs, axis=0)

    aggs = []
    for h in range(_NH):
        out_e = _pass_b(aos[h], ss, qq,
                        p['bn_g'][None], p['bn_b'][None],
                        p['ml_W'], p['ml_b'][None],
                        p['ln_g'][None], p['ln_b'][None])
        _, sc_scatter = _sc_kernels(h * _EH)
        aggs.append(sc_scatter(dst, out_e, zeros))

    return _final(x, aggs[0], aggs[1],
                  params['lc_W'], params['lc_b'][None],
                  params['bnv_g'][None], params['bnv_b'][None])


# speedup vs baseline: 1.2370x; 1.0653x over previous
import functools
import math

import jax
import jax.numpy as jnp
from jax import lax
from jax.experimental import pallas as pl
from jax.experimental.pallas import tpu as pltpu
from jax.experimental.pallas import tpu_sc as plsc

_N = 10000
_E = 160000
_D = 128

_NH = 2
_EH = _E // _NH
_BE = 3200
_NB = _EH // _BE
_NROW = 1000

_NW = 32
_C = 128
_NCHUNK = _EH // _C
_TRIPS = _NCHUNK // _NW
_XTRA = _NCHUNK - _TRIPS * _NW
_STRIPE = 624

_PREC = lax.Precision.DEFAULT


def _silu(v):
    return v * jax.nn.sigmoid(v)


def _rne_bits(x):
    u = lax.bitcast_convert_type(x, jnp.int32)
    return (u + 0x7FFF + ((u >> 16) & 1)) & jnp.int32(-65536)


def _pack2(lo, hi):
    return jnp.bitwise_or((_rne_bits(lo) >> 16) & 65535, _rne_bits(hi))


def _unpack_lo(u):
    return lax.bitcast_convert_type(u << 16, jnp.float32)


def _unpack_hi(u):
    return lax.bitcast_convert_type(u & jnp.int32(-65536), jnp.float32)


def _node_proj_body(x_ref, w_ref, p_ref, q_ref):
    y = jnp.dot(x_ref[...], w_ref[...], precision=_PREC,
                preferred_element_type=jnp.float32)
    p_ref[...] = _pack2(y[:, :256], y[:, 256:512])
    q_ref[...] = y[:, 512:]


def _pass_a_body(ef_ref, s_ref, q_ref, we_ref, b1_ref, kw2_ref, kb2_ref,
                 lw2_ref, lb2_ref, ao_ref, sum_ref, sq_ref):
    t = jnp.dot(ef_ref[...], we_ref[...], precision=_PREC,
                preferred_element_type=jnp.float32)
    su = s_ref[...]
    tk = t[:, :256] + _unpack_lo(su) + b1_ref[:, :256]
    tv = t[:, 256:] + _unpack_hi(su) + b1_ref[:, 256:]
    key = jnp.dot(_silu(tk), kw2_ref[...], precision=_PREC,
                  preferred_element_type=jnp.float32) + kb2_ref[...]
    q = q_ref[...]
    alpha = jnp.concatenate([q, q], axis=1) * key * (1.0 / 16.0)
    out = jnp.dot(_silu(tv), lw2_ref[...], precision=_PREC,
                  preferred_element_type=jnp.float32) + lb2_ref[...]
    ao_ref[...] = _pack2(alpha, out)

    @pl.when(pl.program_id(0) == 0)
    def _():
        sum_ref[...] = jnp.zeros_like(sum_ref)
        sq_ref[...] = jnp.zeros_like(sq_ref)

    sum_ref[...] += jnp.sum(alpha, axis=0, keepdims=True)
    sq_ref[...] += jnp.sum(alpha * alpha, axis=0, keepdims=True)


def _pass_b_body(inv_e, ao_ref, ss_ref, qq_ref, bng_ref,
                 bnb_ref, mlw_ref, mlb_ref, lng_ref, lnb_ref, out_ref):
    mean = jnp.sum(ss_ref[...], axis=0, keepdims=True) * inv_e
    var = jnp.sum(qq_ref[...], axis=0, keepdims=True) * inv_e - mean * mean
    s = bng_ref[...] * lax.rsqrt(var + 1e-5)
    t = bnb_ref[...] - mean * s
    ao = ao_ref[...]
    g = _unpack_hi(ao) * jax.nn.sigmoid(_unpack_lo(ao) * s + t)
    o = jnp.dot(g, mlw_ref[...], precision=_PREC,
                preferred_element_type=jnp.float32) + mlb_ref[...]
    mu = jnp.mean(o, axis=1, keepdims=True)
    v2 = jnp.mean(o * o, axis=1, keepdims=True) - mu * mu
    o = (o - mu) * lax.rsqrt(v2 + 1e-5) * lng_ref[...] + lnb_ref[...]
    out_ref[...] = _silu(o)


def _final_body(n, x_ref, a1_ref, a2_ref, lcw_ref, lcb_ref,
                g_ref, b_ref, out_ref):
    hv = (a1_ref[:n, :] + a1_ref[n:, :]) + (a2_ref[:n, :] + a2_ref[n:, :])
    o = jnp.dot(hv, lcw_ref[...], precision=_PREC,
                preferred_element_type=jnp.float32) + lcb_ref[...]
    mean = jnp.mean(o, axis=0, keepdims=True)
    var = jnp.mean(o * o, axis=0, keepdims=True) - mean * mean
    o = (o - mean) * lax.rsqrt(var + 1e-5) * g_ref[...] + b_ref[...]
    out_ref[...] = x_ref[...] + _silu(o)


def _make_gather_body(e_off):
    def body_fn(src_hbm, dst_hbm, p_hbm, q_hbm, o1_hbm, o2_hbm,
                idx1, rows1, idx2, rows2, sem1, sem2):
        wid = lax.axis_index("c") * 16 + lax.axis_index("s")
        trips = _TRIPS + jnp.where(wid < _XTRA, 1, 0)

        def body(i, carry):
            obase = (wid + _NW * i) * _C
            ibase = e_off + obase
            h1 = pltpu.async_copy(src_hbm.at[pl.ds(ibase, _C)], idx1, sem1)
            h2 = pltpu.async_copy(dst_hbm.at[pl.ds(ibase, _C)], idx2, sem2)
            h1.wait()
            h2.wait()
            g1 = pltpu.async_copy(p_hbm.at[idx1], rows1, sem1)
            g2 = pltpu.async_copy(q_hbm.at[idx2], rows2, sem2)
            g1.wait()
            g2.wait()
            w1 = pltpu.async_copy(rows1, o1_hbm.at[pl.ds(obase, _C)], sem1)
            w2 = pltpu.async_copy(rows2, o2_hbm.at[pl.ds(obase, _C)], sem2)
            w1.wait()
            w2.wait()
            return carry

        lax.fori_loop(0, trips, body, 0)

    return body_fn


def _make_scatter_body(e_off):
    def body_fn(dst_hbm, rows_hbm, zero_hbm, out_hbm, acc, idxv, rowsv,
                sem1, sem2):
        c = lax.axis_index("c")
        s = lax.axis_index("s")
        wid = c * 16 + s
        start = s * _STRIPE

        @pl.when(s < 15)
        def _():
            pltpu.sync_copy(zero_hbm.at[pl.ds(start, _STRIPE)],
                            acc.at[pl.ds(start, _STRIPE)])

        @pl.when(s == 15)
        def _():
            pltpu.sync_copy(zero_hbm.at[pl.ds(15 * _STRIPE, _N - 15 * _STRIPE)],
                            acc.at[pl.ds(15 * _STRIPE, _N - 15 * _STRIPE)])

        plsc.subcore_barrier()
        trips = _TRIPS + jnp.where(wid < _XTRA, 1, 0)

        def body(i, carry):
            obase = (wid + _NW * i) * _C
            h1 = pltpu.async_copy(dst_hbm.at[pl.ds(e_off + obase, _C)], idxv,
                                  sem1)
            h2 = pltpu.async_copy(rows_hbm.at[pl.ds(obase, _C)], rowsv, sem2)
            h1.wait()
            h2.wait()
            pltpu.sync_copy(rowsv, acc.at[idxv], add=True)
            return carry

        lax.fori_loop(0, trips, body, 0)
        plsc.subcore_barrier()

        @pl.when(s < 15)
        def _():
            pltpu.sync_copy(acc.at[pl.ds(start, _STRIPE)],
                            out_hbm.at[pl.ds(c * _N + start, _STRIPE)])

        @pl.when(s == 15)
        def _():
            pltpu.sync_copy(
                acc.at[pl.ds(15 * _STRIPE, _N - 15 * _STRIPE)],
                out_hbm.at[pl.ds(c * _N + 15 * _STRIPE, _N - 15 * _STRIPE)])

    return body_fn


@functools.lru_cache(maxsize=None)
def _sc_kernels(e_off):
    mesh = plsc.VectorSubcoreMesh(core_axis_name="c", subcore_axis_name="s")
    gather = pl.kernel(
        _make_gather_body(e_off), mesh=mesh,
        out_type=[jax.ShapeDtypeStruct((_EH, 256), jnp.int32),
                  jax.ShapeDtypeStruct((_EH, _D), jnp.float32)],
        scratch_types=[pltpu.VMEM((_C,), jnp.int32),
                       pltpu.VMEM((_C, 256), jnp.int32),
                       pltpu.VMEM((_C,), jnp.int32),
                       pltpu.VMEM((_C, _D), jnp.float32),
                       pltpu.SemaphoreType.DMA,
                       pltpu.SemaphoreType.DMA],
    )
    scatter = pl.kernel(
        _make_scatter_body(e_off), mesh=mesh,
        out_type=jax.ShapeDtypeStruct((2 * _N, _D), jnp.float32),
        scratch_types=[pltpu.VMEM_SHARED((_N, _D), jnp.float32),
                       pltpu.VMEM((_C,), jnp.int32),
                       pltpu.VMEM((_C, _D), jnp.float32),
                       pltpu.SemaphoreType.DMA,
                       pltpu.SemaphoreType.DMA],
    )
    return gather, scatter


def _node_proj(x, w):
    return pl.pallas_call(
        _node_proj_body,
        grid=(_N // _NROW,),
        in_specs=[pl.BlockSpec((_NROW, _D), lambda i: (i, 0)),
                  pl.BlockSpec((_D, 640), lambda i: (0, 0))],
        out_specs=[pl.BlockSpec((_NROW, 256), lambda i: (i, 0)),
                   pl.BlockSpec((_NROW, _D), lambda i: (i, 0))],
        out_shape=[jax.ShapeDtypeStruct((_N, 256), jnp.int32),
                   jax.ShapeDtypeStruct((_N, _D), jnp.float32)],
    )(x, w)


def _pass_a(blk_off, ef, s, q, we, b1, kw2, kb2, lw2, lb2):
    return pl.pallas_call(
        _pass_a_body,
        grid=(_NB,),
        in_specs=[pl.BlockSpec((_BE, _D), lambda i: (blk_off + i, 0)),
                  pl.BlockSpec((_BE, 256), lambda i: (i, 0)),
                  pl.BlockSpec((_BE, _D), lambda i: (i, 0)),
                  pl.BlockSpec((_D, 512), lambda i: (0, 0)),
                  pl.BlockSpec((1, 512), lambda i: (0, 0)),
                  pl.BlockSpec((256, 256), lambda i: (0, 0)),
                  pl.BlockSpec((1, 256), lambda i: (0, 0)),
                  pl.BlockSpec((256, 256), lambda i: (0, 0)),
                  pl.BlockSpec((1, 256), lambda i: (0, 0))],
        out_specs=[pl.BlockSpec((_BE, 256), lambda i: (i, 0)),
                   pl.BlockSpec((1, 256), lambda i: (0, 0)),
                   pl.BlockSpec((1, 256), lambda i: (0, 0))],
        out_shape=[jax.ShapeDtypeStruct((_EH, 256), jnp.int32),
                   jax.ShapeDtypeStruct((1, 256), jnp.float32),
                   jax.ShapeDtypeStruct((1, 256), jnp.float32)],
    )(ef, s, q, we, b1, kw2, kb2, lw2, lb2)


def _pass_b(ao, ss, qq, bng, bnb, mlw, mlb, lng, lnb):
    return pl.pallas_call(
        functools.partial(_pass_b_body, 1.0 / _E),
        grid=(_NB,),
        in_specs=[pl.BlockSpec((_BE, 256), lambda i: (i, 0)),
                  pl.BlockSpec((_NH, 256), lambda i: (0, 0)),
                  pl.BlockSpec((_NH, 256), lambda i: (0, 0)),
                  pl.BlockSpec((1, 256), lambda i: (0, 0)),
                  pl.BlockSpec((1, 256), lambda i: (0, 0)),
                  pl.BlockSpec((256, _D), lambda i: (0, 0)),
                  pl.BlockSpec((1, _D), lambda i: (0, 0)),
                  pl.BlockSpec((1, _D), lambda i: (0, 0)),
                  pl.BlockSpec((1, _D), lambda i: (0, 0))],
        out_specs=pl.BlockSpec((_BE, _D), lambda i: (i, 0)),
        out_shape=jax.ShapeDtypeStruct((_EH, _D), jnp.float32),
    )(ao, ss, qq, bng, bnb, mlw, mlb, lng, lnb)


def _final(x, a1, a2, lcw, lcb, g, b):
    return pl.pallas_call(
        functools.partial(_final_body, _N),
        grid=(1,),
        in_specs=[pl.BlockSpec((_N, _D), lambda i: (0, 0)),
                  pl.BlockSpec((2 * _N, _D), lambda i: (0, 0)),
                  pl.BlockSpec((2 * _N, _D), lambda i: (0, 0)),
                  pl.BlockSpec((_D, _D), lambda i: (0, 0)),
                  pl.BlockSpec((1, _D), lambda i: (0, 0)),
                  pl.BlockSpec((1, _D), lambda i: (0, 0)),
                  pl.BlockSpec((1, _D), lambda i: (0, 0))],
        out_specs=pl.BlockSpec((_N, _D), lambda i: (0, 0)),
        out_shape=jax.ShapeDtypeStruct((_N, _D), jnp.float32),
    )(x, a1, a2, lcw, lcb, g, b)


def kernel(x, edge_index, edge_feature, params):
    p = params['heads'][0]
    src = edge_index[0]
    dst = edge_index[1]

    wk = p['K_v2v'] @ p['ku_W1'][:_D]
    wv = p['V_v2v'] @ p['lu_W1'][:_D]
    w_node = jnp.concatenate([wk, wv, p['K_v2v']], axis=1)
    we = jnp.concatenate([p['K_e2v'] @ p['ku_W1'][_D:],
                          p['V_e2v'] @ p['lu_W1'][_D:]], axis=1)
    b1 = jnp.concatenate([p['ku_b1'], p['lu_b1']])[None, :]
    zeros = jnp.zeros((_N, _D), jnp.float32)

    ptab, qtab = _node_proj(x, w_node)

    gathers = []
    for h in range(_NH):
        g, _ = _sc_kernels(h * _EH)
        gathers.append(g(src, dst, ptab, qtab))

    sums = []
    sqs = []
    aos = []
    for h in range(_NH):
        s_e, q_e = gathers[h]
        ao, ssum, ssq = _pass_a(
            h * (_EH // _BE), edge_feature, s_e, q_e, we, b1,
            p['ku_W2'], p['ku_b2'][None], p['lu_W2'], p['lu_b2'][None])
        aos.append(ao)
        sums.append(ssum)
        sqs.append(ssq)
    ss = jnp.concatenate(sums, axis=0)
    qq = jnp.concatenate(sqs, axis=0)

    aggs = []
    for h in range(_NH):
        out_e = _pass_b(aos[h], ss, qq,
                        p['bn_g'][None], p['bn_b'][None],
                        p['ml_W'], p['ml_b'][None],
                        p['ln_g'][None], p['ln_b'][None])
        _, sc_scatter = _sc_kernels(h * _EH)
        aggs.append(sc_scatter(dst, out_e, zeros))

    return _final(x, aggs[0], aggs[1],
                  params['lc_W'], params['lc_b'][None],
                  params['bnv_g'][None], params['bnv_b'][None])


# speedup vs baseline: 1.2465x; 1.0076x over previous
import functools
import math

import jax
import jax.numpy as jnp
from jax import lax
from jax.experimental import pallas as pl
from jax.experimental.pallas import tpu as pltpu
from jax.experimental.pallas import tpu_sc as plsc

_N = 10000
_E = 160000
_D = 128

_NH = 2
_EH = _E // _NH
_BE = 5000
_NB = _EH // _BE
_NROW = 1000

_NW = 32
_C = 128
_NCHUNK = _EH // _C
_TRIPS = _NCHUNK // _NW
_XTRA = _NCHUNK - _TRIPS * _NW
_STRIPE = 624

_PREC = lax.Precision.DEFAULT


def _silu(v):
    return v * jax.nn.sigmoid(v)


def _rne_bits(x):
    u = lax.bitcast_convert_type(x, jnp.int32)
    return (u + 0x7FFF + ((u >> 16) & 1)) & jnp.int32(-65536)


def _pack2(lo, hi):
    return jnp.bitwise_or((_rne_bits(lo) >> 16) & 65535, _rne_bits(hi))


def _unpack_lo(u):
    return lax.bitcast_convert_type(u << 16, jnp.float32)


def _unpack_hi(u):
    return lax.bitcast_convert_type(u & jnp.int32(-65536), jnp.float32)


def _node_proj_body(x_ref, w_ref, p_ref, q_ref):
    y = jnp.dot(x_ref[...], w_ref[...], precision=_PREC,
                preferred_element_type=jnp.float32)
    p_ref[...] = _pack2(y[:, :256], y[:, 256:512])
    q_ref[...] = y[:, 512:]


def _pass_a_body(ef_ref, s_ref, q_ref, we_ref, b1_ref, kw2_ref, kb2_ref,
                 lw2_ref, lb2_ref, ao_ref, sum_ref, sq_ref):
    t = jnp.dot(ef_ref[...], we_ref[...], precision=_PREC,
                preferred_element_type=jnp.float32)
    su = s_ref[...]
    tk = t[:, :256] + _unpack_lo(su) + b1_ref[:, :256]
    tv = t[:, 256:] + _unpack_hi(su) + b1_ref[:, 256:]
    key = jnp.dot(_silu(tk), kw2_ref[...], precision=_PREC,
                  preferred_element_type=jnp.float32) + kb2_ref[...]
    q = q_ref[...]
    alpha = jnp.concatenate([q, q], axis=1) * key * (1.0 / 16.0)
    out = jnp.dot(_silu(tv), lw2_ref[...], precision=_PREC,
                  preferred_element_type=jnp.float32) + lb2_ref[...]
    ao_ref[...] = _pack2(alpha, out)

    @pl.when(pl.program_id(0) == 0)
    def _():
        sum_ref[...] = jnp.zeros_like(sum_ref)
        sq_ref[...] = jnp.zeros_like(sq_ref)

    sum_ref[...] += jnp.sum(alpha, axis=0, keepdims=True)
    sq_ref[...] += jnp.sum(alpha * alpha, axis=0, keepdims=True)


def _pass_b_body(inv_e, ao_ref, ss_ref, qq_ref, bng_ref,
                 bnb_ref, mlw_ref, mlb_ref, lng_ref, lnb_ref, out_ref):
    mean = jnp.sum(ss_ref[...], axis=0, keepdims=True) * inv_e
    var = jnp.sum(qq_ref[...], axis=0, keepdims=True) * inv_e - mean * mean
    s = bng_ref[...] * lax.rsqrt(var + 1e-5)
    t = bnb_ref[...] - mean * s
    ao = ao_ref[...]
    g = _unpack_hi(ao) * jax.nn.sigmoid(_unpack_lo(ao) * s + t)
    o = jnp.dot(g, mlw_ref[...], precision=_PREC,
                preferred_element_type=jnp.float32) + mlb_ref[...]
    mu = jnp.mean(o, axis=1, keepdims=True)
    v2 = jnp.mean(o * o, axis=1, keepdims=True) - mu * mu
    o = (o - mu) * lax.rsqrt(v2 + 1e-5) * lng_ref[...] + lnb_ref[...]
    out_ref[...] = _silu(o)


def _final_body(n, x_ref, a1_ref, a2_ref, lcw_ref, lcb_ref,
                g_ref, b_ref, out_ref):
    hv = (a1_ref[:n, :] + a1_ref[n:, :]) + (a2_ref[:n, :] + a2_ref[n:, :])
    o = jnp.dot(hv, lcw_ref[...], precision=_PREC,
                preferred_element_type=jnp.float32) + lcb_ref[...]
    mean = jnp.mean(o, axis=0, keepdims=True)
    var = jnp.mean(o * o, axis=0, keepdims=True) - mean * mean
    o = (o - mean) * lax.rsqrt(var + 1e-5) * g_ref[...] + b_ref[...]
    out_ref[...] = x_ref[...] + _silu(o)


def _make_gather_body(e_off):
    def body_fn(src_hbm, dst_hbm, p_hbm, q_hbm, o1_hbm, o2_hbm,
                idx1, rows1, idx2, rows2, sem1, sem2):
        wid = lax.axis_index("c") * 16 + lax.axis_index("s")
        trips = _TRIPS + jnp.where(wid < _XTRA, 1, 0)

        def body(i, carry):
            obase = (wid + _NW * i) * _C
            ibase = e_off + obase
            h1 = pltpu.async_copy(src_hbm.at[pl.ds(ibase, _C)], idx1, sem1)
            h2 = pltpu.async_copy(dst_hbm.at[pl.ds(ibase, _C)], idx2, sem2)
            h1.wait()
            h2.wait()
            g1 = pltpu.async_copy(p_hbm.at[idx1], rows1, sem1)
            g2 = pltpu.async_copy(q_hbm.at[idx2], rows2, sem2)
            g1.wait()
            g2.wait()
            w1 = pltpu.async_copy(rows1, o1_hbm.at[pl.ds(obase, _C)], sem1)
            w2 = pltpu.async_copy(rows2, o2_hbm.at[pl.ds(obase, _C)], sem2)
            w1.wait()
            w2.wait()
            return carry

        lax.fori_loop(0, trips, body, 0)

    return body_fn


def _make_scatter_body(e_off):
    def body_fn(dst_hbm, rows_hbm, zero_hbm, out_hbm, acc, idxv, rowsv,
                sem1, sem2):
        c = lax.axis_index("c")
        s = lax.axis_index("s")
        wid = c * 16 + s
        start = s * _STRIPE

        @pl.when(s < 15)
        def _():
            pltpu.sync_copy(zero_hbm.at[pl.ds(start, _STRIPE)],
                            acc.at[pl.ds(start, _STRIPE)])

        @pl.when(s == 15)
        def _():
            pltpu.sync_copy(zero_hbm.at[pl.ds(15 * _STRIPE, _N - 15 * _STRIPE)],
                            acc.at[pl.ds(15 * _STRIPE, _N - 15 * _STRIPE)])

        plsc.subcore_barrier()
        trips = _TRIPS + jnp.where(wid < _XTRA, 1, 0)

        def body(i, carry):
            obase = (wid + _NW * i) * _C
            h1 = pltpu.async_copy(dst_hbm.at[pl.ds(e_off + obase, _C)], idxv,
                                  sem1)
            h2 = pltpu.async_copy(rows_hbm.at[pl.ds(obase, _C)], rowsv, sem2)
            h1.wait()
            h2.wait()
            pltpu.sync_copy(rowsv, acc.at[idxv], add=True)
            return carry

        lax.fori_loop(0, trips, body, 0)
        plsc.subcore_barrier()

        @pl.when(s < 15)
        def _():
            pltpu.sync_copy(acc.at[pl.ds(start, _STRIPE)],
                            out_hbm.at[pl.ds(c * _N + start, _STRIPE)])

        @pl.when(s == 15)
        def _():
            pltpu.sync_copy(
                acc.at[pl.ds(15 * _STRIPE, _N - 15 * _STRIPE)],
                out_hbm.at[pl.ds(c * _N + 15 * _STRIPE, _N - 15 * _STRIPE)])

    return body_fn


@functools.lru_cache(maxsize=None)
def _sc_kernels(e_off):
    mesh = plsc.VectorSubcoreMesh(core_axis_name="c", subcore_axis_name="s")
    gather = pl.kernel(
        _make_gather_body(e_off), mesh=mesh,
        out_type=[jax.ShapeDtypeStruct((_EH, 256), jnp.int32),
                  jax.ShapeDtypeStruct((_EH, _D), jnp.float32)],
        scratch_types=[pltpu.VMEM((_C,), jnp.int32),
                       pltpu.VMEM((_C, 256), jnp.int32),
                       pltpu.VMEM((_C,), jnp.int32),
                       pltpu.VMEM((_C, _D), jnp.float32),
                       pltpu.SemaphoreType.DMA,
                       pltpu.SemaphoreType.DMA],
    )
    scatter = pl.kernel(
        _make_scatter_body(e_off), mesh=mesh,
        out_type=jax.ShapeDtypeStruct((2 * _N, _D), jnp.float32),
        scratch_types=[pltpu.VMEM_SHARED((_N, _D), jnp.float32),
                       pltpu.VMEM((_C,), jnp.int32),
                       pltpu.VMEM((_C, _D), jnp.float32),
                       pltpu.SemaphoreType.DMA,
                       pltpu.SemaphoreType.DMA],
    )
    return gather, scatter


def _node_proj(x, w):
    return pl.pallas_call(
        _node_proj_body,
        grid=(_N // _NROW,),
        in_specs=[pl.BlockSpec((_NROW, _D), lambda i: (i, 0)),
                  pl.BlockSpec((_D, 640), lambda i: (0, 0))],
        out_specs=[pl.BlockSpec((_NROW, 256), lambda i: (i, 0)),
                   pl.BlockSpec((_NROW, _D), lambda i: (i, 0))],
        out_shape=[jax.ShapeDtypeStruct((_N, 256), jnp.int32),
                   jax.ShapeDtypeStruct((_N, _D), jnp.float32)],
    )(x, w)


def _pass_a(blk_off, ef, s, q, we, b1, kw2, kb2, lw2, lb2):
    return pl.pallas_call(
        _pass_a_body,
        grid=(_NB,),
        in_specs=[pl.BlockSpec((_BE, _D), lambda i: (blk_off + i, 0)),
                  pl.BlockSpec((_BE, 256), lambda i: (i, 0)),
                  pl.BlockSpec((_BE, _D), lambda i: (i, 0)),
                  pl.BlockSpec((_D, 512), lambda i: (0, 0)),
                  pl.BlockSpec((1, 512), lambda i: (0, 0)),
                  pl.BlockSpec((256, 256), lambda i: (0, 0)),
                  pl.BlockSpec((1, 256), lambda i: (0, 0)),
                  pl.BlockSpec((256, 256), lambda i: (0, 0)),
                  pl.BlockSpec((1, 256), lambda i: (0, 0))],
        out_specs=[pl.BlockSpec((_BE, 256), lambda i: (i, 0)),
                   pl.BlockSpec((1, 256), lambda i: (0, 0)),
                   pl.BlockSpec((1, 256), lambda i: (0, 0))],
        out_shape=[jax.ShapeDtypeStruct((_EH, 256), jnp.int32),
                   jax.ShapeDtypeStruct((1, 256), jnp.float32),
                   jax.ShapeDtypeStruct((1, 256), jnp.float32)],
    )(ef, s, q, we, b1, kw2, kb2, lw2, lb2)


def _pass_b(ao, ss, qq, bng, bnb, mlw, mlb, lng, lnb):
    return pl.pallas_call(
        functools.partial(_pass_b_body, 1.0 / _E),
        grid=(_NB,),
        in_specs=[pl.BlockSpec((_BE, 256), lambda i: (i, 0)),
                  pl.BlockSpec((_NH, 256), lambda i: (0, 0)),
                  pl.BlockSpec((_NH, 256), lambda i: (0, 0)),
                  pl.BlockSpec((1, 256), lambda i: (0, 0)),
                  pl.BlockSpec((1, 256), lambda i: (0, 0)),
                  pl.BlockSpec((256, _D), lambda i: (0, 0)),
                  pl.BlockSpec((1, _D), lambda i: (0, 0)),
                  pl.BlockSpec((1, _D), lambda i: (0, 0)),
                  pl.BlockSpec((1, _D), lambda i: (0, 0))],
        out_specs=pl.BlockSpec((_BE, _D), lambda i: (i, 0)),
        out_shape=jax.ShapeDtypeStruct((_EH, _D), jnp.float32),
    )(ao, ss, qq, bng, bnb, mlw, mlb, lng, lnb)


def _final(x, a1, a2, lcw, lcb, g, b):
    return pl.pallas_call(
        functools.partial(_final_body, _N),
        grid=(1,),
        in_specs=[pl.BlockSpec((_N, _D), lambda i: (0, 0)),
                  pl.BlockSpec((2 * _N, _D), lambda i: (0, 0)),
                  pl.BlockSpec((2 * _N, _D), lambda i: (0, 0)),
                  pl.BlockSpec((_D, _D), lambda i: (0, 0)),
                  pl.BlockSpec((1, _D), lambda i: (0, 0)),
                  pl.BlockSpec((1, _D), lambda i: (0, 0)),
                  pl.BlockSpec((1, _D), lambda i: (0, 0))],
        out_specs=pl.BlockSpec((_N, _D), lambda i: (0, 0)),
        out_shape=jax.ShapeDtypeStruct((_N, _D), jnp.float32),
    )(x, a1, a2, lcw, lcb, g, b)


def kernel(x, edge_index, edge_feature, params):
    p = params['heads'][0]
    src = edge_index[0]
    dst = edge_index[1]

    wk = p['K_v2v'] @ p['ku_W1'][:_D]
    wv = p['V_v2v'] @ p['lu_W1'][:_D]
    w_node = jnp.concatenate([wk, wv, p['K_v2v']], axis=1)
    we = jnp.concatenate([p['K_e2v'] @ p['ku_W1'][_D:],
                          p['V_e2v'] @ p['lu_W1'][_D:]], axis=1)
    b1 = jnp.concatenate([p['ku_b1'], p['lu_b1']])[None, :]
    zeros = jnp.zeros((_N, _D), jnp.float32)

    ptab, qtab = _node_proj(x, w_node)

    gathers = []
    for h in range(_NH):
        g, _ = _sc_kernels(h * _EH)
        gathers.append(g(src, dst, ptab, qtab))

    sums = []
    sqs = []
    aos = []
    for h in range(_NH):
        s_e, q_e = gathers[h]
        ao, ssum, ssq = _pass_a(
            h * (_EH // _BE), edge_feature, s_e, q_e, we, b1,
            p['ku_W2'], p['ku_b2'][None], p['lu_W2'], p['lu_b2'][None])
        aos.append(ao)
        sums.append(ssum)
        sqs.append(ssq)
    ss = jnp.concatenate(sums, axis=0)
    qq = jnp.concatenate(sqs, axis=0)

    aggs = []
    for h in range(_NH):
        out_e = _pass_b(aos[h], ss, qq,
                        p['bn_g'][None], p['bn_b'][None],
                        p['ml_W'], p['ml_b'][None],
                        p['ln_g'][None], p['ln_b'][None])
        _, sc_scatter = _sc_kernels(h * _EH)
        aggs.append(sc_scatter(dst, out_e, zeros))

    return _final(x, aggs[0], aggs[1],
                  params['lc_W'], params['lc_b'][None],
                  params['bnv_g'][None], params['bnv_b'][None])


# speedup vs baseline: 1.3151x; 1.0551x over previous
import functools
import math

import jax
import jax.numpy as jnp
from jax import lax
from jax.experimental import pallas as pl
from jax.experimental.pallas import tpu as pltpu
from jax.experimental.pallas import tpu_sc as plsc

_N = 10000
_E = 160000
_D = 128

_NH = 2
_EH = _E // _NH
_BE = 5000
_NB = _EH // _BE
_NROW = 1000

_NW = 32
_C = 128
_NCHUNK = _EH // _C
_TRIPS = _NCHUNK // _NW
_XTRA = _NCHUNK - _TRIPS * _NW
_STRIPE = 624

_PREC = lax.Precision.DEFAULT


def _silu(v):
    return v * jax.nn.sigmoid(v)


def _rne_bits(x):
    u = lax.bitcast_convert_type(x, jnp.int32)
    return (u + 0x7FFF + ((u >> 16) & 1)) & jnp.int32(-65536)


def _pack2(lo, hi):
    return jnp.bitwise_or((_rne_bits(lo) >> 16) & 65535, _rne_bits(hi))


def _unpack_lo(u):
    return lax.bitcast_convert_type(u << 16, jnp.float32)


def _unpack_hi(u):
    return lax.bitcast_convert_type(u & jnp.int32(-65536), jnp.float32)


def _node_proj_body(x_ref, w_ref, p_ref, q_ref):
    y = jnp.dot(x_ref[...], w_ref[...], precision=_PREC,
                preferred_element_type=jnp.float32)
    p_ref[...] = _pack2(y[:, :256], y[:, 256:512])
    q_ref[...] = y[:, 512:]


def _pass_a_body(ef_ref, s_ref, q_ref, we_ref, b1_ref, kw2_ref, kb2_ref,
                 lw2_ref, lb2_ref, ao_ref, sum_ref, sq_ref):
    t = jnp.dot(ef_ref[...], we_ref[...], precision=_PREC,
                preferred_element_type=jnp.float32)
    su = s_ref[...]
    tk = t[:, :256] + _unpack_lo(su) + b1_ref[:, :256]
    tv = t[:, 256:] + _unpack_hi(su) + b1_ref[:, 256:]
    key = jnp.dot(_silu(tk), kw2_ref[...], precision=_PREC,
                  preferred_element_type=jnp.float32) + kb2_ref[...]
    q = q_ref[...]
    alpha = jnp.concatenate([q, q], axis=1) * key * (1.0 / 16.0)
    out = jnp.dot(_silu(tv), lw2_ref[...], precision=_PREC,
                  preferred_element_type=jnp.float32) + lb2_ref[...]
    ao_ref[...] = _pack2(alpha, out)

    @pl.when(pl.program_id(0) == 0)
    def _():
        sum_ref[...] = jnp.zeros_like(sum_ref)
        sq_ref[...] = jnp.zeros_like(sq_ref)

    sum_ref[...] += jnp.sum(alpha, axis=0, keepdims=True)
    sq_ref[...] += jnp.sum(alpha * alpha, axis=0, keepdims=True)


def _pass_b_body(inv_e, ao_ref, ss_ref, qq_ref, bng_ref,
                 bnb_ref, mlw_ref, mlb_ref, lng_ref, lnb_ref, out_ref):
    mean = jnp.sum(ss_ref[...], axis=0, keepdims=True) * inv_e
    var = jnp.sum(qq_ref[...], axis=0, keepdims=True) * inv_e - mean * mean
    s = bng_ref[...] * lax.rsqrt(var + 1e-5)
    t = bnb_ref[...] - mean * s
    ao = ao_ref[...]
    g = _unpack_hi(ao) * jax.nn.sigmoid(_unpack_lo(ao) * s + t)
    o = jnp.dot(g, mlw_ref[...], precision=_PREC,
                preferred_element_type=jnp.float32) + mlb_ref[...]
    mu = jnp.mean(o, axis=1, keepdims=True)
    v2 = jnp.mean(o * o, axis=1, keepdims=True) - mu * mu
    o = (o - mu) * lax.rsqrt(v2 + 1e-5) * lng_ref[...] + lnb_ref[...]
    out_ref[...] = _silu(o)


def _final_body(n, x_ref, a1_ref, a2_ref, lcw_ref, lcb_ref,
                g_ref, b_ref, out_ref):
    hv = (a1_ref[:n, :] + a1_ref[n:, :]) + (a2_ref[:n, :] + a2_ref[n:, :])
    o = jnp.dot(hv, lcw_ref[...], precision=_PREC,
                preferred_element_type=jnp.float32) + lcb_ref[...]
    mean = jnp.mean(o, axis=0, keepdims=True)
    var = jnp.mean(o * o, axis=0, keepdims=True) - mean * mean
    o = (o - mean) * lax.rsqrt(var + 1e-5) * g_ref[...] + b_ref[...]
    out_ref[...] = x_ref[...] + _silu(o)


def _make_gather_body(e_off):
    def body_fn(src_hbm, dst_hbm, p_hbm, q_hbm, o1_hbm, o2_hbm,
                idx1a, idx1b, idx2a, idx2b, rows1a, rows1b, rows2a, rows2b,
                si1, si2, sg1, sg2, sw1, sw2):
        wid = lax.axis_index("c") * 16 + lax.axis_index("s")
        trips = _TRIPS + jnp.where(wid < _XTRA, 1, 0)

        def ibase_of(i):
            return e_off + (wid + _NW * i) * _C

        pltpu.async_copy(src_hbm.at[pl.ds(ibase_of(0), _C)], idx1a, si1)
        pltpu.async_copy(dst_hbm.at[pl.ds(ibase_of(0), _C)], idx2a, si2)

        def step(i, idx1c, idx2c, rows1c, rows2c, idx1n, idx2n, rows1n, rows2n):
            obase = (wid + _NW * i) * _C
            pltpu.make_async_copy(src_hbm.at[pl.ds(ibase_of(0), _C)],
                                  idx1c, si1).wait()
            pltpu.make_async_copy(dst_hbm.at[pl.ds(ibase_of(0), _C)],
                                  idx2c, si2).wait()
            g1 = pltpu.async_copy(p_hbm.at[idx1c], rows1c, sg1)
            g2 = pltpu.async_copy(q_hbm.at[idx2c], rows2c, sg2)

            @pl.when(i + 1 < trips)
            def _():
                pltpu.async_copy(src_hbm.at[pl.ds(ibase_of(i + 1), _C)],
                                 idx1n, si1)
                pltpu.async_copy(dst_hbm.at[pl.ds(ibase_of(i + 1), _C)],
                                 idx2n, si2)

            g1.wait()
            g2.wait()

            @pl.when(i > 0)
            def _():
                pltpu.make_async_copy(rows1n, o1_hbm.at[pl.ds(0, _C)],
                                      sw1).wait()
                pltpu.make_async_copy(rows2n, o2_hbm.at[pl.ds(0, _C)],
                                      sw2).wait()

            pltpu.async_copy(rows1c, o1_hbm.at[pl.ds(obase, _C)], sw1)
            pltpu.async_copy(rows2c, o2_hbm.at[pl.ds(obase, _C)], sw2)

        def body(i, carry):
            @pl.when(i % 2 == 0)
            def _():
                step(i, idx1a, idx2a, rows1a, rows2a,
                     idx1b, idx2b, rows1b, rows2b)

            @pl.when(i % 2 == 1)
            def _():
                step(i, idx1b, idx2b, rows1b, rows2b,
                     idx1a, idx2a, rows1a, rows2a)

            return carry

        lax.fori_loop(0, trips, body, 0)
        pltpu.make_async_copy(rows1a, o1_hbm.at[pl.ds(0, _C)], sw1).wait()
        pltpu.make_async_copy(rows2a, o2_hbm.at[pl.ds(0, _C)], sw2).wait()

    return body_fn


def _make_scatter_body(e_off):
    def body_fn(dst_hbm, rows_hbm, zero_hbm, out_hbm, acc, idxva, idxvb,
                rowsva, rowsvb, sem1, sem2):
        c = lax.axis_index("c")
        s = lax.axis_index("s")
        wid = c * 16 + s
        start = s * _STRIPE

        @pl.when(s < 15)
        def _():
            pltpu.sync_copy(zero_hbm.at[pl.ds(start, _STRIPE)],
                            acc.at[pl.ds(start, _STRIPE)])

        @pl.when(s == 15)
        def _():
            pltpu.sync_copy(zero_hbm.at[pl.ds(15 * _STRIPE, _N - 15 * _STRIPE)],
                            acc.at[pl.ds(15 * _STRIPE, _N - 15 * _STRIPE)])

        plsc.subcore_barrier()
        trips = _TRIPS + jnp.where(wid < _XTRA, 1, 0)

        pltpu.async_copy(dst_hbm.at[pl.ds(e_off + wid * _C, _C)], idxva, sem1)
        pltpu.async_copy(rows_hbm.at[pl.ds(wid * _C, _C)], rowsva, sem2)

        def step(i, idxc, rowsc, idxn, rowsn):
            pltpu.make_async_copy(dst_hbm.at[pl.ds(e_off, _C)], idxc,
                                  sem1).wait()
            pltpu.make_async_copy(rows_hbm.at[pl.ds(0, _C)], rowsc,
                                  sem2).wait()

            @pl.when(i + 1 < trips)
            def _():
                obn = (wid + _NW * (i + 1)) * _C
                pltpu.async_copy(dst_hbm.at[pl.ds(e_off + obn, _C)], idxn,
                                 sem1)
                pltpu.async_copy(rows_hbm.at[pl.ds(obn, _C)], rowsn, sem2)

            pltpu.sync_copy(rowsc, acc.at[idxc], add=True)

        def body(i, carry):
            @pl.when(i % 2 == 0)
            def _():
                step(i, idxva, rowsva, idxvb, rowsvb)

            @pl.when(i % 2 == 1)
            def _():
                step(i, idxvb, rowsvb, idxva, rowsva)

            return carry

        lax.fori_loop(0, trips, body, 0)
        plsc.subcore_barrier()

        @pl.when(s < 15)
        def _():
            pltpu.sync_copy(acc.at[pl.ds(start, _STRIPE)],
                            out_hbm.at[pl.ds(c * _N + start, _STRIPE)])

        @pl.when(s == 15)
        def _():
            pltpu.sync_copy(
                acc.at[pl.ds(15 * _STRIPE, _N - 15 * _STRIPE)],
                out_hbm.at[pl.ds(c * _N + 15 * _STRIPE, _N - 15 * _STRIPE)])

    return body_fn


@functools.lru_cache(maxsize=None)
def _sc_kernels(e_off):
    mesh = plsc.VectorSubcoreMesh(core_axis_name="c", subcore_axis_name="s")
    gather = pl.kernel(
        _make_gather_body(e_off), mesh=mesh,
        out_type=[jax.ShapeDtypeStruct((_EH, 256), jnp.int32),
                  jax.ShapeDtypeStruct((_EH, _D), jnp.float32)],
        scratch_types=[pltpu.VMEM((_C,), jnp.int32),
                       pltpu.VMEM((_C,), jnp.int32),
                       pltpu.VMEM((_C,), jnp.int32),
                       pltpu.VMEM((_C,), jnp.int32),
                       pltpu.VMEM((_C, 256), jnp.int32),
                       pltpu.VMEM((_C, 256), jnp.int32),
                       pltpu.VMEM((_C, _D), jnp.float32),
                       pltpu.VMEM((_C, _D), jnp.float32),
                       pltpu.SemaphoreType.DMA,
                       pltpu.SemaphoreType.DMA,
                       pltpu.SemaphoreType.DMA,
                       pltpu.SemaphoreType.DMA,
                       pltpu.SemaphoreType.DMA,
                       pltpu.SemaphoreType.DMA],
    )
    scatter = pl.kernel(
        _make_scatter_body(e_off), mesh=mesh,
        out_type=jax.ShapeDtypeStruct((2 * _N, _D), jnp.float32),
        scratch_types=[pltpu.VMEM_SHARED((_N, _D), jnp.float32),
                       pltpu.VMEM((_C,), jnp.int32),
                       pltpu.VMEM((_C,), jnp.int32),
                       pltpu.VMEM((_C, _D), jnp.float32),
                       pltpu.VMEM((_C, _D), jnp.float32),
                       pltpu.SemaphoreType.DMA,
                       pltpu.SemaphoreType.DMA],
    )
    return gather, scatter


def _node_proj(x, w):
    return pl.pallas_call(
        _node_proj_body,
        grid=(_N // _NROW,),
        in_specs=[pl.BlockSpec((_NROW, _D), lambda i: (i, 0)),
                  pl.BlockSpec((_D, 640), lambda i: (0, 0))],
        out_specs=[pl.BlockSpec((_NROW, 256), lambda i: (i, 0)),
                   pl.BlockSpec((_NROW, _D), lambda i: (i, 0))],
        out_shape=[jax.ShapeDtypeStruct((_N, 256), jnp.int32),
                   jax.ShapeDtypeStruct((_N, _D), jnp.float32)],
    )(x, w)


def _pass_a(blk_off, ef, s, q, we, b1, kw2, kb2, lw2, lb2):
    return pl.pallas_call(
        _pass_a_body,
        grid=(_NB,),
        in_specs=[pl.BlockSpec((_BE, _D), lambda i: (blk_off + i, 0)),
                  pl.BlockSpec((_BE, 256), lambda i: (i, 0)),
                  pl.BlockSpec((_BE, _D), lambda i: (i, 0)),
                  pl.BlockSpec((_D, 512), lambda i: (0, 0)),
                  pl.BlockSpec((1, 512), lambda i: (0, 0)),
                  pl.BlockSpec((256, 256), lambda i: (0, 0)),
                  pl.BlockSpec((1, 256), lambda i: (0, 0)),
                  pl.BlockSpec((256, 256), lambda i: (0, 0)),
                  pl.BlockSpec((1, 256), lambda i: (0, 0))],
        out_specs=[pl.BlockSpec((_BE, 256), lambda i: (i, 0)),
                   pl.BlockSpec((1, 256), lambda i: (0, 0)),
                   pl.BlockSpec((1, 256), lambda i: (0, 0))],
        out_shape=[jax.ShapeDtypeStruct((_EH, 256), jnp.int32),
                   jax.ShapeDtypeStruct((1, 256), jnp.float32),
                   jax.ShapeDtypeStruct((1, 256), jnp.float32)],
    )(ef, s, q, we, b1, kw2, kb2, lw2, lb2)


def _pass_b(ao, ss, qq, bng, bnb, mlw, mlb, lng, lnb):
    return pl.pallas_call(
        functools.partial(_pass_b_body, 1.0 / _E),
        grid=(_NB,),
        in_specs=[pl.BlockSpec((_BE, 256), lambda i: (i, 0)),
                  pl.BlockSpec((_NH, 256), lambda i: (0, 0)),
                  pl.BlockSpec((_NH, 256), lambda i: (0, 0)),
                  pl.BlockSpec((1, 256), lambda i: (0, 0)),
                  pl.BlockSpec((1, 256), lambda i: (0, 0)),
                  pl.BlockSpec((256, _D), lambda i: (0, 0)),
                  pl.BlockSpec((1, _D), lambda i: (0, 0)),
                  pl.BlockSpec((1, _D), lambda i: (0, 0)),
                  pl.BlockSpec((1, _D), lambda i: (0, 0))],
        out_specs=pl.BlockSpec((_BE, _D), lambda i: (i, 0)),
        out_shape=jax.ShapeDtypeStruct((_EH, _D), jnp.float32),
    )(ao, ss, qq, bng, bnb, mlw, mlb, lng, lnb)


def _final(x, a1, a2, lcw, lcb, g, b):
    return pl.pallas_call(
        functools.partial(_final_body, _N),
        grid=(1,),
        in_specs=[pl.BlockSpec((_N, _D), lambda i: (0, 0)),
                  pl.BlockSpec((2 * _N, _D), lambda i: (0, 0)),
                  pl.BlockSpec((2 * _N, _D), lambda i: (0, 0)),
                  pl.BlockSpec((_D, _D), lambda i: (0, 0)),
                  pl.BlockSpec((1, _D), lambda i: (0, 0)),
                  pl.BlockSpec((1, _D), lambda i: (0, 0)),
                  pl.BlockSpec((1, _D), lambda i: (0, 0))],
        out_specs=pl.BlockSpec((_N, _D), lambda i: (0, 0)),
        out_shape=jax.ShapeDtypeStruct((_N, _D), jnp.float32),
    )(x, a1, a2, lcw, lcb, g, b)


def kernel(x, edge_index, edge_feature, params):
    p = params['heads'][0]
    src = edge_index[0]
    dst = edge_index[1]

    wk = p['K_v2v'] @ p['ku_W1'][:_D]
    wv = p['V_v2v'] @ p['lu_W1'][:_D]
    w_node = jnp.concatenate([wk, wv, p['K_v2v']], axis=1)
    we = jnp.concatenate([p['K_e2v'] @ p['ku_W1'][_D:],
                          p['V_e2v'] @ p['lu_W1'][_D:]], axis=1)
    b1 = jnp.concatenate([p['ku_b1'], p['lu_b1']])[None, :]
    zeros = jnp.zeros((_N, _D), jnp.float32)

    ptab, qtab = _node_proj(x, w_node)

    gathers = []
    for h in range(_NH):
        g, _ = _sc_kernels(h * _EH)
        gathers.append(g(src, dst, ptab, qtab))

    sums = []
    sqs = []
    aos = []
    for h in range(_NH):
        s_e, q_e = gathers[h]
        ao, ssum, ssq = _pass_a(
            h * (_EH // _BE), edge_feature, s_e, q_e, we, b1,
            p['ku_W2'], p['ku_b2'][None], p['lu_W2'], p['lu_b2'][None])
        aos.append(ao)
        sums.append(ssum)
        sqs.append(ssq)
    ss = jnp.concatenate(sums, axis=0)
    qq = jnp.concatenate(sqs, axis=0)

    aggs = []
    for h in range(_NH):
        out_e = _pass_b(aos[h], ss, qq,
                        p['bn_g'][None], p['bn_b'][None],
                        p['ml_W'], p['ml_b'][None],
                        p['ln_g'][None], p['ln_b'][None])
        _, sc_scatter = _sc_kernels(h * _EH)
        aggs.append(sc_scatter(dst, out_e, zeros))

    return _final(x, aggs[0], aggs[1],
                  params['lc_W'], params['lc_b'][None],
                  params['bnv_g'][None], params['bnv_b'][None])
